# baseline (speedup 1.0000x reference)
import functools

import jax
import jax.numpy as jnp
from jax import lax
from jax.experimental import pallas as pl
from jax.experimental.pallas import tpu as pltpu
from jax.experimental.pallas import tpu_sc as plsc

N = 10000
E = 320000
D = 128
H = 32
G = 64
C = 96

NPAD = 10240
ROWS_W = NPAD // 16
CH = 128
NCHUNK = 160
EPS = NCHUNK * CH
EPAD = EPS * 16

_mesh = plsc.VectorSubcoreMesh(core_axis_name="c", subcore_axis_name="s")
_sc_params = pltpu.CompilerParams(use_tc_tiling_on_sc=False)


@functools.partial(
    pl.kernel,
    out_type=jax.ShapeDtypeStruct((2, NPAD), jnp.float32),
    mesh=_mesh,
    scratch_types=[
        pltpu.VMEM((NCHUNK, CH), jnp.int32),
        pltpu.VMEM((CH,), jnp.float32),
        pltpu.VMEM((ROWS_W,), jnp.float32),
        pltpu.VMEM_SHARED((NPAD,), jnp.float32),
        pltpu.SemaphoreType.DMA,
    ],
    compiler_params=_sc_params,
)
def _deg_kernel(dst_hbm, deg_hbm, dst_v, ones_v, zeros_v, deg_sp, sem):
    cid = lax.axis_index("c")
    sid = lax.axis_index("s")
    for i in range(CH // 16):
        ones_v[pl.ds(i * 16, 16)] = jnp.full((16,), 1.0, jnp.float32)
    for i in range(ROWS_W // 16):
        zeros_v[pl.ds(i * 16, 16)] = jnp.zeros((16,), jnp.float32)
    pltpu.sync_copy(zeros_v, deg_sp.at[pl.ds(sid * ROWS_W, ROWS_W)])
    pltpu.sync_copy(dst_hbm.at[cid, sid], dst_v)
    plsc.subcore_barrier()

    K = 8

    def body(b, carry):
        for j in range(K):
            pltpu.async_copy(ones_v, deg_sp.at[dst_v.at[b * K + j]], sem,
                             add=True)
        for j in range(K):
            pltpu.make_async_copy(ones_v, deg_sp.at[dst_v.at[b * K + j]],
                                  sem).wait()
        return carry

    lax.fori_loop(0, NCHUNK // K, body, 0)
    plsc.subcore_barrier()
    pltpu.sync_copy(deg_sp.at[pl.ds(sid * ROWS_W, ROWS_W)],
                    deg_hbm.at[cid, pl.ds(sid * ROWS_W, ROWS_W)])


@functools.partial(
    pl.kernel,
    out_type=jax.ShapeDtypeStruct((2, NPAD, H), jnp.float32),
    mesh=_mesh,
    scratch_types=[
        pltpu.VMEM((NCHUNK, CH), jnp.int32),
        pltpu.VMEM((NCHUNK, CH), jnp.int32),
        pltpu.VMEM((CH, H), jnp.float32),
        pltpu.VMEM((CH, H), jnp.float32),
        pltpu.VMEM_SHARED((NPAD, H), jnp.float32),
        pltpu.SemaphoreType.DMA,
        pltpu.SemaphoreType.DMA,
    ],
    compiler_params=_sc_params,
)
def _agg_kernel(u_hbm, src_hbm, dst_hbm, acc_hbm, src_v, dst_v, rows0_v,
                rows1_v, acc_sp, sem0, sem1):
    cid = lax.axis_index("c")
    sid = lax.axis_index("s")
    for r in range(CH):
        for j in range(H // 16):
            rows0_v[r, pl.ds(j * 16, 16)] = jnp.zeros((16,), jnp.float32)
    for k in range(ROWS_W // CH):
        pltpu.sync_copy(rows0_v, acc_sp.at[pl.ds(sid * ROWS_W + k * CH, CH)])
    pltpu.sync_copy(src_hbm.at[cid, sid], src_v)
    pltpu.sync_copy(dst_hbm.at[cid, sid], dst_v)
    plsc.subcore_barrier()

    pltpu.async_copy(u_hbm.at[src_v.at[0]], rows0_v, sem0)
    last = NCHUNK // 2 - 1

    def body(k, carry):
        i0 = 2 * k
        i1 = i0 + 1
        i2 = lax.select(k < last, i0 + 2, 0)
        pltpu.make_async_copy(u_hbm.at[src_v.at[i0]], rows0_v, sem0).wait()
        pltpu.async_copy(u_hbm.at[src_v.at[i1]], rows1_v, sem1)
        pltpu.sync_copy(rows0_v, acc_sp.at[dst_v.at[i0]], add=True)
        pltpu.make_async_copy(u_hbm.at[src_v.at[i1]], rows1_v, sem1).wait()
        pltpu.async_copy(u_hbm.at[src_v.at[i2]], rows0_v, sem0)
        pltpu.sync_copy(rows1_v, acc_sp.at[dst_v.at[i1]], add=True)
        return carry

    lax.fori_loop(0, NCHUNK // 2, body, 0)
    pltpu.make_async_copy(u_hbm.at[src_v.at[0]], rows0_v, sem0).wait()
    plsc.subcore_barrier()
    pltpu.sync_copy(acc_sp.at[pl.ds(sid * ROWS_W, ROWS_W)],
                    acc_hbm.at[cid, pl.ds(sid * ROWS_W, ROWS_W)])


def _mid_body(x_ref, w1_ref, deg_ref, u_ref, dinv_ref):
    d = deg_ref[...] + 1.0
    dv = lax.rsqrt(d)
    xw = jnp.dot(x_ref[...], w1_ref[...], preferred_element_type=jnp.float32)
    u_ref[...] = xw * dv
    dinv_ref[...] = dv


def _tc_mid(X, W1, deg_flat):
    blk = 2048
    grid = (2 * NPAD) // blk
    return pl.pallas_call(
        _mid_body,
        grid=(grid,),
        in_specs=[
            pl.BlockSpec((blk, D), lambda i: (i, 0)),
            pl.BlockSpec((D, H), lambda i: (0, 0)),
            pl.BlockSpec((blk, 1), lambda i: (i, 0)),
        ],
        out_specs=[
            pl.BlockSpec((blk, H), lambda i: (i, 0)),
            pl.BlockSpec((blk, 1), lambda i: (i, 0)),
        ],
        out_shape=[
            jax.ShapeDtypeStruct((2 * NPAD, H), jnp.float32),
            jax.ShapeDtypeStruct((2 * NPAD, 1), jnp.float32),
        ],
    )(X, W1, deg_flat)


def _final_body(acc_ref, u_ref, dinv_ref, b1_ref, batch_ref, w2_ref, b2_ref,
                out_ref):
    embs = []
    u_all = u_ref[...]
    dv_all = dinv_ref[...]
    for c in range(2):
        a_c = acc_ref[c]
        u_c = u_all[c * NPAD:(c + 1) * NPAD]
        dv_c = dv_all[c * NPAD:(c + 1) * NPAD]
        h = jnp.tanh(dv_c * (a_c + u_c) + b1_ref[...])
        b_c = batch_ref[c]
        iota = lax.broadcasted_iota(jnp.int32, (NPAD, G), 1)
        M = (b_c == iota).astype(jnp.float32)
        sums = lax.dot_general(M, h, (((0,), (0,)), ((), ())),
                               preferred_element_type=jnp.float32)
        ones = jnp.ones((NPAD, 1), jnp.float32)
        cnt = lax.dot_general(M, ones, (((0,), (0,)), ((), ())),
                              preferred_element_type=jnp.float32)
        pooled = sums / jnp.maximum(cnt, 1.0)
        embs.append(jnp.tanh(pooled))
    embedding = jnp.concatenate(embs, axis=1)
    out_ref[...] = (jnp.dot(embedding, w2_ref[...],
                            preferred_element_type=jnp.float32) + b2_ref[...])


def _tc_final(acc, U, dinv, b1, batch2, W2, b2):
    return pl.pallas_call(
        _final_body,
        out_shape=jax.ShapeDtypeStruct((G, C), jnp.float32),
    )(acc, U, dinv, b1.reshape(1, H), batch2, W2, b2.reshape(1, C))


def _pad_edges(ei, offset):
    src = jnp.concatenate([ei[0], jnp.full((EPAD - E,), N, jnp.int32)])
    dst = jnp.concatenate([ei[1], jnp.full((EPAD - E,), N, jnp.int32)])
    return (src + offset).reshape(16, NCHUNK, CH), dst.reshape(16, NCHUNK, CH)


def kernel(x_s, edge_index_s, x_s_batch, x_t, edge_index_t, x_t_batch, y, W1,
           b1, W2, b2):
    zrows = jnp.zeros((NPAD - N, D), jnp.float32)
    X = jnp.concatenate([x_s, zrows, x_t, zrows])

    src_s, dst_s = _pad_edges(edge_index_s, 0)
    src_t, dst_t = _pad_edges(edge_index_t, NPAD)
    src_comb = jnp.stack([src_s, src_t])
    dst_comb = jnp.stack([dst_s, dst_t])

    bpad = jnp.full((NPAD - N,), G, jnp.int32)
    batch2 = jnp.stack([jnp.concatenate([x_s_batch, bpad]),
                        jnp.concatenate([x_t_batch, bpad])])[..., None]

    deg = _deg_kernel(dst_comb)
    U, dinv = _tc_mid(X, W1, deg.reshape(2 * NPAD, 1))
    acc = _agg_kernel(U, src_comb, dst_comb)
    return _tc_final(acc, U, dinv, b1, batch2, W2, b2)


# speedup vs baseline: 1.4889x; 1.4889x over previous
import functools

import jax
import jax.numpy as jnp
from jax import lax
from jax.experimental import pallas as pl
from jax.experimental.pallas import tpu as pltpu
from jax.experimental.pallas import tpu_sc as plsc

N = 10000
E = 320000
D = 128
H = 32
G = 64
C = 96

NPAD = 10240
ROWS_W = NPAD // 16
CH = 128
NCHUNK = 160
EPS = NCHUNK * CH
EPAD = EPS * 16

_mesh = plsc.VectorSubcoreMesh(core_axis_name="c", subcore_axis_name="s")
_sc_params = pltpu.CompilerParams(use_tc_tiling_on_sc=False)


@functools.partial(
    pl.kernel,
    out_type=jax.ShapeDtypeStruct((2, NPAD), jnp.float32),
    mesh=_mesh,
    scratch_types=[
        pltpu.VMEM((NCHUNK, CH), jnp.int32),
        pltpu.VMEM((CH,), jnp.float32),
        pltpu.VMEM((ROWS_W,), jnp.float32),
        pltpu.VMEM_SHARED((NPAD,), jnp.float32),
    ],
    compiler_params=_sc_params,
)
def _deg_kernel(dst_hbm, deg_hbm, dst_v, ones_v, zeros_v, deg_sp):
    cid = lax.axis_index("c")
    sid = lax.axis_index("s")
    for i in range(CH // 16):
        ones_v[pl.ds(i * 16, 16)] = jnp.full((16,), 1.0, jnp.float32)
    for i in range(ROWS_W // 16):
        zeros_v[pl.ds(i * 16, 16)] = jnp.zeros((16,), jnp.float32)
    pltpu.sync_copy(zeros_v, deg_sp.at[pl.ds(sid * ROWS_W, ROWS_W)])
    pltpu.sync_copy(dst_hbm.at[cid, sid], dst_v)
    plsc.subcore_barrier()

    def body(i, carry):
        pltpu.sync_copy(ones_v, deg_sp.at[dst_v.at[i]], add=True)
        return carry

    lax.fori_loop(0, NCHUNK, body, 0)
    plsc.subcore_barrier()
    pltpu.sync_copy(deg_sp.at[pl.ds(sid * ROWS_W, ROWS_W)],
                    deg_hbm.at[cid, pl.ds(sid * ROWS_W, ROWS_W)])


@functools.partial(
    pl.kernel,
    out_type=jax.ShapeDtypeStruct((2, NPAD, H), jnp.float32),
    mesh=_mesh,
    scratch_types=[
        pltpu.VMEM((NCHUNK, CH), jnp.int32),
        pltpu.VMEM((NCHUNK, CH), jnp.int32),
        pltpu.VMEM((CH, H), jnp.float32),
        pltpu.VMEM_SHARED((NPAD, H), jnp.float32),
        pltpu.VMEM_SHARED((NPAD, H), jnp.float32),
        pltpu.SemaphoreType.DMA,
    ],
    compiler_params=_sc_params,
)
def _agg_kernel(u_hbm, src_hbm, dst_hbm, acc_hbm, src_v, dst_v, rows_v,
                u_sp, acc_sp, sem):
    cid = lax.axis_index("c")
    sid = lax.axis_index("s")
    for r in range(CH):
        for j in range(H // 16):
            rows_v[r, pl.ds(j * 16, 16)] = jnp.zeros((16,), jnp.float32)
    for k in range(ROWS_W // CH):
        pltpu.sync_copy(rows_v, acc_sp.at[pl.ds(sid * ROWS_W + k * CH, CH)])
    pltpu.sync_copy(u_hbm.at[cid, pl.ds(sid * ROWS_W, ROWS_W)],
                    u_sp.at[pl.ds(sid * ROWS_W, ROWS_W)])
    pltpu.sync_copy(src_hbm.at[cid, sid], src_v)
    pltpu.sync_copy(dst_hbm.at[cid, sid], dst_v)
    plsc.subcore_barrier()

    def body(i, carry):
        pltpu.async_copy(u_sp.at[src_v.at[i]], rows_v, sem).wait()
        pltpu.sync_copy(rows_v, acc_sp.at[dst_v.at[i]], add=True)
        return carry

    lax.fori_loop(0, NCHUNK, body, 0)
    plsc.subcore_barrier()
    pltpu.sync_copy(acc_sp.at[pl.ds(sid * ROWS_W, ROWS_W)],
                    acc_hbm.at[cid, pl.ds(sid * ROWS_W, ROWS_W)])


def _mid_body(x_ref, w1_ref, deg_ref, u_ref, dinv_ref):
    d = deg_ref[...] + 1.0
    dv = lax.rsqrt(d)
    xw = jnp.dot(x_ref[...], w1_ref[...], preferred_element_type=jnp.float32)
    u_ref[...] = xw * dv
    dinv_ref[...] = dv


def _tc_mid(X, W1, deg_flat):
    blk = 2048
    grid = (2 * NPAD) // blk
    return pl.pallas_call(
        _mid_body,
        grid=(grid,),
        in_specs=[
            pl.BlockSpec((blk, D), lambda i: (i, 0)),
            pl.BlockSpec((D, H), lambda i: (0, 0)),
            pl.BlockSpec((blk, 1), lambda i: (i, 0)),
        ],
        out_specs=[
            pl.BlockSpec((blk, H), lambda i: (i, 0)),
            pl.BlockSpec((blk, 1), lambda i: (i, 0)),
        ],
        out_shape=[
            jax.ShapeDtypeStruct((2 * NPAD, H), jnp.float32),
            jax.ShapeDtypeStruct((2 * NPAD, 1), jnp.float32),
        ],
    )(X, W1, deg_flat)


def _final_body(acc_ref, u_ref, dinv_ref, b1_ref, batch_ref, w2_ref, b2_ref,
                out_ref):
    embs = []
    dv_all = dinv_ref[...]
    for c in range(2):
        a_c = acc_ref[c]
        u_c = u_ref[c]
        dv_c = dv_all[c * NPAD:(c + 1) * NPAD]
        h = jnp.tanh(dv_c * (a_c + u_c) + b1_ref[...])
        b_c = batch_ref[c]
        iota = lax.broadcasted_iota(jnp.int32, (NPAD, G), 1)
        M = (b_c == iota).astype(jnp.float32)
        sums = lax.dot_general(M, h, (((0,), (0,)), ((), ())),
                               preferred_element_type=jnp.float32)
        ones = jnp.ones((NPAD, 1), jnp.float32)
        cnt = lax.dot_general(M, ones, (((0,), (0,)), ((), ())),
                              preferred_element_type=jnp.float32)
        pooled = sums / jnp.maximum(cnt, 1.0)
        embs.append(jnp.tanh(pooled))
    embedding = jnp.concatenate(embs, axis=1)
    out_ref[...] = (jnp.dot(embedding, w2_ref[...],
                            preferred_element_type=jnp.float32) + b2_ref[...])


def _tc_final(acc, U, dinv, b1, batch2, W2, b2):
    return pl.pallas_call(
        _final_body,
        out_shape=jax.ShapeDtypeStruct((G, C), jnp.float32),
    )(acc, U, dinv, b1.reshape(1, H), batch2, W2, b2.reshape(1, C))


def _pad_edges(ei, offset):
    src = jnp.concatenate([ei[0], jnp.full((EPAD - E,), N, jnp.int32)])
    dst = jnp.concatenate([ei[1], jnp.full((EPAD - E,), N, jnp.int32)])
    return (src + offset).reshape(16, NCHUNK, CH), dst.reshape(16, NCHUNK, CH)


def kernel(x_s, edge_index_s, x_s_batch, x_t, edge_index_t, x_t_batch, y, W1,
           b1, W2, b2):
    zrows = jnp.zeros((NPAD - N, D), jnp.float32)
    X = jnp.concatenate([x_s, zrows, x_t, zrows])

    src_s, dst_s = _pad_edges(edge_index_s, 0)
    src_t, dst_t = _pad_edges(edge_index_t, 0)
    src_comb = jnp.stack([src_s, src_t])
    dst_comb = jnp.stack([dst_s, dst_t])

    bpad = jnp.full((NPAD - N,), G, jnp.int32)
    batch2 = jnp.stack([jnp.concatenate([x_s_batch, bpad]),
                        jnp.concatenate([x_t_batch, bpad])])[..., None]

    deg = _deg_kernel(dst_comb)
    U, dinv = _tc_mid(X, W1, deg.reshape(2 * NPAD, 1))
    U2 = U.reshape(2, NPAD, H)
    acc = _agg_kernel(U2, src_comb, dst_comb)
    return _tc_final(acc, U2, dinv, b1, batch2, W2, b2)


# speedup vs baseline: 1.6606x; 1.1153x over previous
import functools

import jax
import jax.numpy as jnp
from jax import lax
from jax.experimental import pallas as pl
from jax.experimental.pallas import tpu as pltpu
from jax.experimental.pallas import tpu_sc as plsc

N = 10000
E = 320000
D = 128
H = 32
G = 64
C = 96

NPAD = 10240
ROWS_W = NPAD // 16
CH = 128
NCHUNK = 160
EPS = NCHUNK * CH
EPAD = EPS * 16

_mesh = plsc.VectorSubcoreMesh(core_axis_name="c", subcore_axis_name="s")
_sc_params = pltpu.CompilerParams(use_tc_tiling_on_sc=False)


@functools.partial(
    pl.kernel,
    out_type=jax.ShapeDtypeStruct((2, NPAD), jnp.float32),
    mesh=_mesh,
    scratch_types=[
        pltpu.VMEM((NCHUNK, CH), jnp.int32),
        pltpu.VMEM((CH,), jnp.float32),
        pltpu.VMEM((ROWS_W,), jnp.float32),
        pltpu.VMEM_SHARED((NPAD,), jnp.float32),
    ],
    compiler_params=_sc_params,
)
def _deg_kernel(dst_hbm, deg_hbm, dst_v, ones_v, zeros_v, deg_sp):
    cid = lax.axis_index("c")
    sid = lax.axis_index("s")
    for i in range(CH // 16):
        ones_v[pl.ds(i * 16, 16)] = jnp.full((16,), 1.0, jnp.float32)
    for i in range(ROWS_W // 16):
        zeros_v[pl.ds(i * 16, 16)] = jnp.zeros((16,), jnp.float32)
    pltpu.sync_copy(zeros_v, deg_sp.at[pl.ds(sid * ROWS_W, ROWS_W)])
    pltpu.sync_copy(dst_hbm.at[cid, sid], dst_v)
    plsc.subcore_barrier()

    def body(i, carry):
        pltpu.sync_copy(ones_v, deg_sp.at[dst_v.at[i]], add=True)
        return carry

    lax.fori_loop(0, NCHUNK, body, 0)
    plsc.subcore_barrier()
    pltpu.sync_copy(deg_sp.at[pl.ds(sid * ROWS_W, ROWS_W)],
                    deg_hbm.at[cid, pl.ds(sid * ROWS_W, ROWS_W)])


@functools.partial(
    pl.kernel,
    out_type=jax.ShapeDtypeStruct((2, NPAD, H), jnp.float32),
    mesh=_mesh,
    scratch_types=[
        pltpu.VMEM((NCHUNK, CH), jnp.int32),
        pltpu.VMEM((NCHUNK, CH), jnp.int32),
        pltpu.VMEM((2, CH, H), jnp.float32),
        pltpu.VMEM_SHARED((NPAD, H), jnp.float32),
        pltpu.VMEM_SHARED((NPAD, H), jnp.float32),
        pltpu.SemaphoreType.DMA,
        pltpu.SemaphoreType.DMA,
    ],
    compiler_params=_sc_params,
)
def _agg_kernel(u_hbm, src_hbm, dst_hbm, acc_hbm, src_v, dst_v, rows_v,
                u_sp, acc_sp, g0, g1):
    cid = lax.axis_index("c")
    sid = lax.axis_index("s")
    for r in range(CH):
        for j in range(H // 16):
            rows_v[0, r, pl.ds(j * 16, 16)] = jnp.zeros((16,), jnp.float32)
    for k in range(ROWS_W // CH):
        pltpu.sync_copy(rows_v.at[0],
                        acc_sp.at[pl.ds(sid * ROWS_W + k * CH, CH)])
    pltpu.sync_copy(u_hbm.at[cid, pl.ds(sid * ROWS_W, ROWS_W)],
                    u_sp.at[pl.ds(sid * ROWS_W, ROWS_W)])
    pltpu.sync_copy(src_hbm.at[cid, sid], src_v)
    pltpu.sync_copy(dst_hbm.at[cid, sid], dst_v)
    plsc.subcore_barrier()

    pltpu.async_copy(u_sp.at[src_v.at[0]], rows_v.at[0], g0)
    last = NCHUNK // 2 - 1

    def body(k, carry):
        i0 = 2 * k
        i1 = i0 + 1
        i2 = lax.select(k < last, i0 + 2, 0)
        pltpu.make_async_copy(u_sp.at[src_v.at[i0]], rows_v.at[0], g0).wait()
        pltpu.async_copy(u_sp.at[src_v.at[i1]], rows_v.at[1], g1)
        pltpu.sync_copy(rows_v.at[0], acc_sp.at[dst_v.at[i0]], add=True)
        pltpu.make_async_copy(u_sp.at[src_v.at[i1]], rows_v.at[1], g1).wait()
        pltpu.async_copy(u_sp.at[src_v.at[i2]], rows_v.at[0], g0)
        pltpu.sync_copy(rows_v.at[1], acc_sp.at[dst_v.at[i1]], add=True)
        return carry

    lax.fori_loop(0, NCHUNK // 2, body, 0)
    pltpu.make_async_copy(u_sp.at[src_v.at[0]], rows_v.at[0], g0).wait()
    plsc.subcore_barrier()
    pltpu.sync_copy(acc_sp.at[pl.ds(sid * ROWS_W, ROWS_W)],
                    acc_hbm.at[cid, pl.ds(sid * ROWS_W, ROWS_W)])


def _mid_body(x_ref, w1_ref, deg_ref, u_ref, dinv_ref):
    d = deg_ref[...] + 1.0
    dv = lax.rsqrt(d)
    xw = jnp.dot(x_ref[...], w1_ref[...], preferred_element_type=jnp.float32)
    u_ref[...] = xw * dv
    dinv_ref[...] = dv


def _tc_mid(X, W1, deg_flat):
    blk = 2048
    grid = (2 * NPAD) // blk
    return pl.pallas_call(
        _mid_body,
        grid=(grid,),
        in_specs=[
            pl.BlockSpec((blk, D), lambda i: (i, 0)),
            pl.BlockSpec((D, H), lambda i: (0, 0)),
            pl.BlockSpec((blk, 1), lambda i: (i, 0)),
        ],
        out_specs=[
            pl.BlockSpec((blk, H), lambda i: (i, 0)),
            pl.BlockSpec((blk, 1), lambda i: (i, 0)),
        ],
        out_shape=[
            jax.ShapeDtypeStruct((2 * NPAD, H), jnp.float32),
            jax.ShapeDtypeStruct((2 * NPAD, 1), jnp.float32),
        ],
    )(X, W1, deg_flat)


def _final_body(acc_ref, u_ref, dinv_ref, b1_ref, batch_ref, w2_ref, b2_ref,
                out_ref):
    embs = []
    dv_all = dinv_ref[...]
    for c in range(2):
        a_c = acc_ref[c]
        u_c = u_ref[c]
        dv_c = dv_all[c * NPAD:(c + 1) * NPAD]
        h = jnp.tanh(dv_c * (a_c + u_c) + b1_ref[...])
        b_c = batch_ref[c]
        iota = lax.broadcasted_iota(jnp.int32, (NPAD, G), 1)
        M = (b_c == iota).astype(jnp.float32)
        sums = lax.dot_general(M, h, (((0,), (0,)), ((), ())),
                               preferred_element_type=jnp.float32)
        ones = jnp.ones((NPAD, 1), jnp.float32)
        cnt = lax.dot_general(M, ones, (((0,), (0,)), ((), ())),
                              preferred_element_type=jnp.float32)
        pooled = sums / jnp.maximum(cnt, 1.0)
        embs.append(jnp.tanh(pooled))
    embedding = jnp.concatenate(embs, axis=1)
    out_ref[...] = (jnp.dot(embedding, w2_ref[...],
                            preferred_element_type=jnp.float32) + b2_ref[...])


def _tc_final(acc, U, dinv, b1, batch2, W2, b2):
    return pl.pallas_call(
        _final_body,
        out_shape=jax.ShapeDtypeStruct((G, C), jnp.float32),
    )(acc, U, dinv, b1.reshape(1, H), batch2, W2, b2.reshape(1, C))


def _pad_edges(ei, offset):
    src = jnp.concatenate([ei[0], jnp.full((EPAD - E,), N, jnp.int32)])
    dst = jnp.concatenate([ei[1], jnp.full((EPAD - E,), N, jnp.int32)])
    return (src + offset).reshape(16, NCHUNK, CH), dst.reshape(16, NCHUNK, CH)


def kernel(x_s, edge_index_s, x_s_batch, x_t, edge_index_t, x_t_batch, y, W1,
           b1, W2, b2):
    zrows = jnp.zeros((NPAD - N, D), jnp.float32)
    X = jnp.concatenate([x_s, zrows, x_t, zrows])

    src_s, dst_s = _pad_edges(edge_index_s, 0)
    src_t, dst_t = _pad_edges(edge_index_t, 0)
    src_comb = jnp.stack([src_s, src_t])
    dst_comb = jnp.stack([dst_s, dst_t])

    bpad = jnp.full((NPAD - N,), G, jnp.int32)
    batch2 = jnp.stack([jnp.concatenate([x_s_batch, bpad]),
                        jnp.concatenate([x_t_batch, bpad])])[..., None]

    deg = _deg_kernel(dst_comb)
    U, dinv = _tc_mid(X, W1, deg.reshape(2 * NPAD, 1))
    U2 = U.reshape(2, NPAD, H)
    acc = _agg_kernel(U2, src_comb, dst_comb)
    return _tc_final(acc, U2, dinv, b1, batch2, W2, b2)


# speedup vs baseline: 1.7339x; 1.0441x over previous
import functools

import jax
import jax.numpy as jnp
from jax import lax
from jax.experimental import pallas as pl
from jax.experimental.pallas import tpu as pltpu
from jax.experimental.pallas import tpu_sc as plsc

N = 10000
E = 320000
D = 128
H = 32
G = 64
C = 96

NPAD = 10240
ROWS_W = NPAD // 16
CH = 128
NCHUNK = 160
EPS = NCHUNK * CH
EPAD = EPS * 16

_mesh = plsc.VectorSubcoreMesh(core_axis_name="c", subcore_axis_name="s")
_sc_params = pltpu.CompilerParams(use_tc_tiling_on_sc=False)


@functools.partial(
    pl.kernel,
    out_type=jax.ShapeDtypeStruct((2, NPAD), jnp.float32),
    mesh=_mesh,
    scratch_types=[
        pltpu.VMEM((NCHUNK, CH), jnp.int32),
        pltpu.VMEM((CH,), jnp.float32),
        pltpu.VMEM((ROWS_W,), jnp.float32),
        pltpu.VMEM_SHARED((NPAD,), jnp.float32),
        pltpu.SemaphoreType.DMA,
    ],
    compiler_params=_sc_params,
)
def _deg_kernel(dst_hbm, deg_hbm, dst_v, ones_v, zeros_v, deg_sp, sem):
    cid = lax.axis_index("c")
    sid = lax.axis_index("s")
    for i in range(CH // 16):
        ones_v[pl.ds(i * 16, 16)] = jnp.full((16,), 1.0, jnp.float32)
    for i in range(ROWS_W // 16):
        zeros_v[pl.ds(i * 16, 16)] = jnp.zeros((16,), jnp.float32)
    pltpu.sync_copy(zeros_v, deg_sp.at[pl.ds(sid * ROWS_W, ROWS_W)])
    pltpu.sync_copy(dst_hbm.at[cid, sid], dst_v)
    plsc.subcore_barrier()

    K = 8

    def body(b, carry):
        for j in range(K):
            pltpu.async_copy(ones_v, deg_sp.at[dst_v.at[b * K + j]], sem,
                             add=True)
        for j in range(K):
            pltpu.make_async_copy(ones_v, deg_sp.at[dst_v.at[b * K + j]],
                                  sem).wait()
        return carry

    lax.fori_loop(0, NCHUNK // K, body, 0)
    plsc.subcore_barrier()
    pltpu.sync_copy(deg_sp.at[pl.ds(sid * ROWS_W, ROWS_W)],
                    deg_hbm.at[cid, pl.ds(sid * ROWS_W, ROWS_W)])


@functools.partial(
    pl.kernel,
    out_type=jax.ShapeDtypeStruct((2, NPAD, H), jnp.float32),
    mesh=_mesh,
    scratch_types=[
        pltpu.VMEM((NCHUNK, CH), jnp.int32),
        pltpu.VMEM((NCHUNK, CH), jnp.int32),
        pltpu.VMEM((4, CH, H), jnp.float32),
        pltpu.VMEM_SHARED((NPAD, H), jnp.float32),
        pltpu.VMEM_SHARED((NPAD, H), jnp.float32),
        pltpu.SemaphoreType.DMA,
        pltpu.SemaphoreType.DMA,
        pltpu.SemaphoreType.DMA,
        pltpu.SemaphoreType.DMA,
    ],
    compiler_params=_sc_params,
)
def _agg_kernel(u_hbm, src_hbm, dst_hbm, acc_hbm, src_v, dst_v, rows_v,
                u_sp, acc_sp, g0, g1, g2, g3):
    cid = lax.axis_index("c")
    sid = lax.axis_index("s")
    for r in range(CH):
        for j in range(H // 16):
            rows_v[0, r, pl.ds(j * 16, 16)] = jnp.zeros((16,), jnp.float32)
    for k in range(ROWS_W // CH):
        pltpu.sync_copy(rows_v.at[0],
                        acc_sp.at[pl.ds(sid * ROWS_W + k * CH, CH)])
    pltpu.sync_copy(u_hbm.at[cid, pl.ds(sid * ROWS_W, ROWS_W)],
                    u_sp.at[pl.ds(sid * ROWS_W, ROWS_W)])
    pltpu.sync_copy(src_hbm.at[cid, sid], src_v)
    pltpu.sync_copy(dst_hbm.at[cid, sid], dst_v)
    plsc.subcore_barrier()

    gsems = [g0, g1, g2, g3]
    for j in range(4):
        pltpu.async_copy(u_sp.at[src_v.at[j]], rows_v.at[j], gsems[j])

    def body(k, carry):
        base = 4 * k
        for j in range(4):
            i = base + j
            nxt = lax.rem(i + 4, NCHUNK)
            pltpu.make_async_copy(u_sp.at[src_v.at[i]], rows_v.at[j],
                                  gsems[j]).wait()
            pltpu.sync_copy(rows_v.at[j], acc_sp.at[dst_v.at[i]], add=True)
            pltpu.async_copy(u_sp.at[src_v.at[nxt]], rows_v.at[j], gsems[j])
        return carry

    lax.fori_loop(0, NCHUNK // 4, body, 0)
    for j in range(4):
        pltpu.make_async_copy(u_sp.at[src_v.at[j]], rows_v.at[j],
                              gsems[j]).wait()
    plsc.subcore_barrier()
    pltpu.sync_copy(acc_sp.at[pl.ds(sid * ROWS_W, ROWS_W)],
                    acc_hbm.at[cid, pl.ds(sid * ROWS_W, ROWS_W)])


def _mid_body(x_ref, w1_ref, deg_ref, u_ref, dinv_ref):
    d = deg_ref[...] + 1.0
    dv = lax.rsqrt(d)
    xw = jnp.dot(x_ref[...], w1_ref[...], preferred_element_type=jnp.float32)
    u_ref[...] = xw * dv
    dinv_ref[...] = dv


def _tc_mid(X, W1, deg_flat):
    blk = 2048
    grid = (2 * NPAD) // blk
    return pl.pallas_call(
        _mid_body,
        grid=(grid,),
        in_specs=[
            pl.BlockSpec((blk, D), lambda i: (i, 0)),
            pl.BlockSpec((D, H), lambda i: (0, 0)),
            pl.BlockSpec((blk, 1), lambda i: (i, 0)),
        ],
        out_specs=[
            pl.BlockSpec((blk, H), lambda i: (i, 0)),
            pl.BlockSpec((blk, 1), lambda i: (i, 0)),
        ],
        out_shape=[
            jax.ShapeDtypeStruct((2 * NPAD, H), jnp.float32),
            jax.ShapeDtypeStruct((2 * NPAD, 1), jnp.float32),
        ],
    )(X, W1, deg_flat)


def _final_body(acc_ref, u_ref, dinv_ref, b1_ref, batch_ref, w2_ref, b2_ref,
                out_ref):
    embs = []
    dv_all = dinv_ref[...]
    for c in range(2):
        a_c = acc_ref[c]
        u_c = u_ref[c]
        dv_c = dv_all[c * NPAD:(c + 1) * NPAD]
        h = jnp.tanh(dv_c * (a_c + u_c) + b1_ref[...])
        b_c = batch_ref[c]
        iota = lax.broadcasted_iota(jnp.int32, (NPAD, G), 1)
        M = (b_c == iota).astype(jnp.float32)
        sums = lax.dot_general(M, h, (((0,), (0,)), ((), ())),
                               preferred_element_type=jnp.float32)
        ones = jnp.ones((NPAD, 1), jnp.float32)
        cnt = lax.dot_general(M, ones, (((0,), (0,)), ((), ())),
                              preferred_element_type=jnp.float32)
        pooled = sums / jnp.maximum(cnt, 1.0)
        embs.append(jnp.tanh(pooled))
    embedding = jnp.concatenate(embs, axis=1)
    out_ref[...] = (jnp.dot(embedding, w2_ref[...],
                            preferred_element_type=jnp.float32) + b2_ref[...])


def _tc_final(acc, U, dinv, b1, batch2, W2, b2):
    return pl.pallas_call(
        _final_body,
        out_shape=jax.ShapeDtypeStruct((G, C), jnp.float32),
    )(acc, U, dinv, b1.reshape(1, H), batch2, W2, b2.reshape(1, C))


def _pad_edges(ei, offset):
    src = jnp.concatenate([ei[0], jnp.full((EPAD - E,), N, jnp.int32)])
    dst = jnp.concatenate([ei[1], jnp.full((EPAD - E,), N, jnp.int32)])
    return (src + offset).reshape(16, NCHUNK, CH), dst.reshape(16, NCHUNK, CH)


def kernel(x_s, edge_index_s, x_s_batch, x_t, edge_index_t, x_t_batch, y, W1,
           b1, W2, b2):
    zrows = jnp.zeros((NPAD - N, D), jnp.float32)
    X = jnp.concatenate([x_s, zrows, x_t, zrows])

    src_s, dst_s = _pad_edges(edge_index_s, 0)
    src_t, dst_t = _pad_edges(edge_index_t, 0)
    src_comb = jnp.stack([src_s, src_t])
    dst_comb = jnp.stack([dst_s, dst_t])

    bpad = jnp.full((NPAD - N,), G, jnp.int32)
    batch2 = jnp.stack([jnp.concatenate([x_s_batch, bpad]),
                        jnp.concatenate([x_t_batch, bpad])])[..., None]

    deg = _deg_kernel(dst_comb)
    U, dinv = _tc_mid(X, W1, deg.reshape(2 * NPAD, 1))
    U2 = U.reshape(2, NPAD, H)
    acc = _agg_kernel(U2, src_comb, dst_comb)
    return _tc_final(acc, U2, dinv, b1, batch2, W2, b2)


# speedup vs baseline: 1.7356x; 1.0010x over previous
import functools

import jax
import jax.numpy as jnp
from jax import lax
from jax.experimental import pallas as pl
from jax.experimental.pallas import tpu as pltpu
from jax.experimental.pallas import tpu_sc as plsc

N = 10000
E = 320000
D = 128
H = 32
G = 64
C = 96

NPAD = 10240
ROWS_W = NPAD // 16
CH = 128
NCHUNK = 160
EPS = NCHUNK * CH
EPAD = EPS * 16

_mesh = plsc.VectorSubcoreMesh(core_axis_name="c", subcore_axis_name="s")
_sc_params = pltpu.CompilerParams(use_tc_tiling_on_sc=False,
                                  needs_layout_passes=False)


def _rsqrt16(x):
    i = plsc.bitcast(x, jnp.int32)
    i = jnp.int32(0x5F3759DF) - lax.shift_right_logical(i, 1)
    y = plsc.bitcast(i, jnp.float32)
    for _ in range(3):
        y = y * (1.5 - 0.5 * x * y * y)
    return y


@functools.partial(
    pl.kernel,
    out_type=[
        jax.ShapeDtypeStruct((2, NPAD, H), jnp.float32),
        jax.ShapeDtypeStruct((2, NPAD), jnp.float32),
    ],
    mesh=_mesh,
    scratch_types=[
        pltpu.VMEM((NCHUNK, CH), jnp.int32),
        pltpu.VMEM((NCHUNK, CH), jnp.int32),
        pltpu.VMEM((4, CH, H), jnp.float32),
        pltpu.VMEM((ROWS_W,), jnp.float32),
        pltpu.VMEM((ROWS_W,), jnp.float32),
        pltpu.VMEM((ROWS_W, H), jnp.float32),
        pltpu.VMEM((CH,), jnp.float32),
        pltpu.VMEM_SHARED((NPAD,), jnp.float32),
        pltpu.VMEM_SHARED((NPAD, H), jnp.float32),
        pltpu.VMEM_SHARED((NPAD, H), jnp.float32),
        pltpu.SemaphoreType.DMA,
        pltpu.SemaphoreType.DMA,
        pltpu.SemaphoreType.DMA,
        pltpu.SemaphoreType.DMA,
    ],
    compiler_params=_sc_params,
)
def _mega_kernel(xw_hbm, src_hbm, dst_hbm, acc_hbm, dinv_hbm, src_v, dst_v,
                 rows_v, deg_v, dinv_v, xw_v, ones_v, deg_sp, u_sp, acc_sp,
                 g0, g1, g2, g3):
    cid = lax.axis_index("c")
    sid = lax.axis_index("s")
    for i in range(CH // 16):
        ones_v[pl.ds(i * 16, 16)] = jnp.full((16,), 1.0, jnp.float32)
    for r in range(CH):
        for j in range(H // 16):
            rows_v[0, r, pl.ds(j * 16, 16)] = jnp.zeros((16,), jnp.float32)
    for i in range(ROWS_W // 16):
        deg_v[pl.ds(i * 16, 16)] = jnp.zeros((16,), jnp.float32)
    pltpu.sync_copy(deg_v, deg_sp.at[pl.ds(sid * ROWS_W, ROWS_W)])
    for k in range(ROWS_W // CH):
        pltpu.sync_copy(rows_v.at[0],
                        acc_sp.at[pl.ds(sid * ROWS_W + k * CH, CH)])
    pltpu.sync_copy(dst_hbm.at[cid, sid], dst_v)
    plsc.subcore_barrier()

    K = 8

    def deg_body(b, carry):
        for j in range(K):
            pltpu.async_copy(ones_v, deg_sp.at[dst_v.at[b * K + j]], g0,
                             add=True)
        for j in range(K):
            pltpu.make_async_copy(ones_v, deg_sp.at[dst_v.at[b * K + j]],
                                  g0).wait()
        return carry

    lax.fori_loop(0, NCHUNK // K, deg_body, 0)
    pltpu.sync_copy(src_hbm.at[cid, sid], src_v)
    pltpu.sync_copy(xw_hbm.at[cid, pl.ds(sid * ROWS_W, ROWS_W)], xw_v)
    plsc.subcore_barrier()

    pltpu.sync_copy(deg_sp.at[pl.ds(sid * ROWS_W, ROWS_W)], deg_v)
    for i in range(ROWS_W // 16):
        dinv_v[pl.ds(i * 16, 16)] = _rsqrt16(
            deg_v[pl.ds(i * 16, 16)] + 1.0)

    def scale_body(i, carry):
        base = i * 16
        dv16 = dinv_v[pl.ds(base, 16)]
        for k in range(16):
            dv = dv16[k]
            for j in range(H // 16):
                xw_v[base + k, pl.ds(j * 16, 16)] = (
                    xw_v[base + k, pl.ds(j * 16, 16)] * dv)
        return carry

    lax.fori_loop(0, ROWS_W // 16, scale_body, 0)
    pltpu.sync_copy(dinv_v, dinv_hbm.at[cid, pl.ds(sid * ROWS_W, ROWS_W)])
    pltpu.sync_copy(xw_v, u_sp.at[pl.ds(sid * ROWS_W, ROWS_W)])
    plsc.subcore_barrier()

    gsems = [g0, g1, g2, g3]
    for j in range(4):
        pltpu.async_copy(u_sp.at[src_v.at[j]], rows_v.at[j], gsems[j])

    def body(k, carry):
        base = 4 * k
        for j in range(4):
            i = base + j
            nxt = lax.rem(i + 4, NCHUNK)
            pltpu.make_async_copy(u_sp.at[src_v.at[i]], rows_v.at[j],
                                  gsems[j]).wait()
            pltpu.sync_copy(rows_v.at[j], acc_sp.at[dst_v.at[i]], add=True)
            pltpu.async_copy(u_sp.at[src_v.at[nxt]], rows_v.at[j], gsems[j])
        return carry

    lax.fori_loop(0, NCHUNK // 4, body, 0)
    for j in range(4):
        pltpu.make_async_copy(u_sp.at[src_v.at[j]], rows_v.at[j],
                              gsems[j]).wait()
    plsc.subcore_barrier()
    pltpu.sync_copy(acc_sp.at[pl.ds(sid * ROWS_W, ROWS_W)],
                    acc_hbm.at[cid, pl.ds(sid * ROWS_W, ROWS_W)])


def _xw_body(x_ref, w1_ref, xw_ref):
    xw_ref[...] = jnp.dot(x_ref[...], w1_ref[...],
                          preferred_element_type=jnp.float32)


def _tc_xw(X, W1):
    blk = 2048
    grid = (2 * NPAD) // blk
    return pl.pallas_call(
        _xw_body,
        grid=(grid,),
        in_specs=[
            pl.BlockSpec((blk, D), lambda i: (i, 0)),
            pl.BlockSpec((D, H), lambda i: (0, 0)),
        ],
        out_specs=pl.BlockSpec((blk, H), lambda i: (i, 0)),
        out_shape=jax.ShapeDtypeStruct((2 * NPAD, H), jnp.float32),
    )(X, W1)


def _final_body(acc_ref, xw_ref, dinv_ref, b1_ref, batch_ref, w2_ref, b2_ref,
                out_ref):
    embs = []
    for c in range(2):
        a_c = acc_ref[c]
        dv_c = dinv_ref[c][:, None]
        u_c = xw_ref[c] * dv_c
        h = jnp.tanh(dv_c * (a_c + u_c) + b1_ref[...])
        b_c = batch_ref[c]
        iota = lax.broadcasted_iota(jnp.int32, (NPAD, G), 1)
        M = (b_c == iota).astype(jnp.float32)
        sums = lax.dot_general(M, h, (((0,), (0,)), ((), ())),
                               preferred_element_type=jnp.float32)
        ones = jnp.ones((NPAD, 1), jnp.float32)
        cnt = lax.dot_general(M, ones, (((0,), (0,)), ((), ())),
                              preferred_element_type=jnp.float32)
        pooled = sums / jnp.maximum(cnt, 1.0)
        embs.append(jnp.tanh(pooled))
    embedding = jnp.concatenate(embs, axis=1)
    out_ref[...] = (jnp.dot(embedding, w2_ref[...],
                            preferred_element_type=jnp.float32) + b2_ref[...])


def _tc_final(acc, XW2, dinv, b1, batch2, W2, b2):
    return pl.pallas_call(
        _final_body,
        out_shape=jax.ShapeDtypeStruct((G, C), jnp.float32),
    )(acc, XW2, dinv, b1.reshape(1, H), batch2, W2, b2.reshape(1, C))


def _pad_edges(ei, offset):
    src = jnp.concatenate([ei[0], jnp.full((EPAD - E,), N, jnp.int32)])
    dst = jnp.concatenate([ei[1], jnp.full((EPAD - E,), N, jnp.int32)])
    return (src + offset).reshape(16, NCHUNK, CH), dst.reshape(16, NCHUNK, CH)


def kernel(x_s, edge_index_s, x_s_batch, x_t, edge_index_t, x_t_batch, y, W1,
           b1, W2, b2):
    zrows = jnp.zeros((NPAD - N, D), jnp.float32)
    X = jnp.concatenate([x_s, zrows, x_t, zrows])

    src_s, dst_s = _pad_edges(edge_index_s, 0)
    src_t, dst_t = _pad_edges(edge_index_t, 0)
    src_comb = jnp.stack([src_s, src_t])
    dst_comb = jnp.stack([dst_s, dst_t])

    bpad = jnp.full((NPAD - N,), G, jnp.int32)
    batch2 = jnp.stack([jnp.concatenate([x_s_batch, bpad]),
                        jnp.concatenate([x_t_batch, bpad])])[..., None]

    XW2 = _tc_xw(X, W1).reshape(2, NPAD, H)
    acc, dinv = _mega_kernel(XW2, src_comb, dst_comb)
    return _tc_final(acc, XW2, dinv, b1, batch2, W2, b2)


# speedup vs baseline: 1.9785x; 1.1399x over previous
import functools

import jax
import jax.numpy as jnp
from jax import lax
from jax.experimental import pallas as pl
from jax.experimental.pallas import tpu as pltpu
from jax.experimental.pallas import tpu_sc as plsc

N = 10000
E = 320000
D = 128
H = 32
G = 64
C = 96

NPAD = 10240
ROWS_W = NPAD // 16
CH = 128
NCHUNK = 160
EPS = NCHUNK * CH
EPAD = EPS * 16

_mesh = plsc.VectorSubcoreMesh(core_axis_name="c", subcore_axis_name="s")
_sc_params = pltpu.CompilerParams(use_tc_tiling_on_sc=False,
                                  needs_layout_passes=False)


def _rsqrt16(x):
    i = plsc.bitcast(x, jnp.int32)
    i = jnp.int32(0x5F3759DF) - lax.shift_right_logical(i, 1)
    y = plsc.bitcast(i, jnp.float32)
    for _ in range(3):
        y = y * (1.5 - 0.5 * x * y * y)
    return y


@functools.partial(
    pl.kernel,
    out_type=jax.ShapeDtypeStruct((2, NPAD), jnp.float32),
    mesh=_mesh,
    scratch_types=[
        pltpu.VMEM((NCHUNK, CH), jnp.int32),
        pltpu.VMEM((CH,), jnp.float32),
        pltpu.VMEM((ROWS_W,), jnp.float32),
        pltpu.VMEM_SHARED((NPAD,), jnp.float32),
        pltpu.SemaphoreType.DMA,
    ],
    compiler_params=_sc_params,
)
def _deg_kernel(dst_hbm, deg_hbm, dst_v, ones_v, zeros_v, deg_sp, sem):
    cid = lax.axis_index("c")
    sid = lax.axis_index("s")
    for i in range(CH // 16):
        ones_v[pl.ds(i * 16, 16)] = jnp.full((16,), 1.0, jnp.float32)
    for i in range(ROWS_W // 16):
        zeros_v[pl.ds(i * 16, 16)] = jnp.zeros((16,), jnp.float32)
    pltpu.sync_copy(zeros_v, deg_sp.at[pl.ds(sid * ROWS_W, ROWS_W)])
    pltpu.sync_copy(dst_hbm.at[cid, sid], dst_v)
    plsc.subcore_barrier()

    K = 8

    def body(b, carry):
        for j in range(K):
            pltpu.async_copy(ones_v, deg_sp.at[dst_v.at[b * K + j]], sem,
                             add=True)
        for j in range(K):
            pltpu.make_async_copy(ones_v, deg_sp.at[dst_v.at[b * K + j]],
                                  sem).wait()
        return carry

    lax.fori_loop(0, NCHUNK // K, body, 0)
    plsc.subcore_barrier()
    pltpu.sync_copy(deg_sp.at[pl.ds(sid * ROWS_W, ROWS_W)],
                    deg_hbm.at[cid, pl.ds(sid * ROWS_W, ROWS_W)])


@functools.partial(
    pl.kernel,
    out_type=[
        jax.ShapeDtypeStruct((2, NPAD, H), jnp.float32),
        jax.ShapeDtypeStruct((2, NPAD), jnp.float32),
    ],
    mesh=_mesh,
    scratch_types=[
        pltpu.VMEM((NCHUNK, CH), jnp.int32),
        pltpu.VMEM((NCHUNK, CH), jnp.int32),
        pltpu.VMEM((4, CH, H), jnp.float32),
        pltpu.VMEM((ROWS_W,), jnp.float32),
        pltpu.VMEM((ROWS_W,), jnp.float32),
        pltpu.VMEM((ROWS_W, H), jnp.float32),
        pltpu.VMEM_SHARED((NPAD, H), jnp.float32),
        pltpu.VMEM_SHARED((NPAD, H), jnp.float32),
        pltpu.SemaphoreType.DMA,
        pltpu.SemaphoreType.DMA,
        pltpu.SemaphoreType.DMA,
        pltpu.SemaphoreType.DMA,
    ],
    compiler_params=_sc_params,
)
def _mega_kernel(xw_hbm, src_hbm, dst_hbm, deg_hbm, acc_hbm, dinv_hbm, src_v,
                 dst_v, rows_v, deg_v, dinv_v, xw_v, u_sp, acc_sp,
                 g0, g1, g2, g3):
    cid = lax.axis_index("c")
    sid = lax.axis_index("s")
    for r in range(CH):
        for j in range(H // 16):
            rows_v[0, r, pl.ds(j * 16, 16)] = jnp.zeros((16,), jnp.float32)
    for k in range(ROWS_W // CH):
        pltpu.sync_copy(rows_v.at[0],
                        acc_sp.at[pl.ds(sid * ROWS_W + k * CH, CH)])
    pltpu.sync_copy(dst_hbm.at[cid, sid], dst_v)
    pltpu.sync_copy(src_hbm.at[cid, sid], src_v)
    pltpu.sync_copy(xw_hbm.at[cid, pl.ds(sid * ROWS_W, ROWS_W)], xw_v)

    pltpu.sync_copy(deg_hbm.at[cid, pl.ds(sid * ROWS_W, ROWS_W)], deg_v)
    for i in range(ROWS_W // 16):
        dinv_v[pl.ds(i * 16, 16)] = _rsqrt16(
            deg_v[pl.ds(i * 16, 16)] + 1.0)

    def scale_body(i, carry):
        base = i * 16
        dv16 = dinv_v[pl.ds(base, 16)]
        for k in range(16):
            dv = dv16[k]
            for j in range(H // 16):
                xw_v[base + k, pl.ds(j * 16, 16)] = (
                    xw_v[base + k, pl.ds(j * 16, 16)] * dv)
        return carry

    lax.fori_loop(0, ROWS_W // 16, scale_body, 0)
    pltpu.sync_copy(dinv_v, dinv_hbm.at[cid, pl.ds(sid * ROWS_W, ROWS_W)])
    pltpu.sync_copy(xw_v, u_sp.at[pl.ds(sid * ROWS_W, ROWS_W)])
    plsc.subcore_barrier()

    gsems = [g0, g1, g2, g3]
    for j in range(4):
        pltpu.async_copy(u_sp.at[src_v.at[j]], rows_v.at[j], gsems[j])

    def body(k, carry):
        base = 4 * k
        for j in range(4):
            i = base + j
            nxt = lax.rem(i + 4, NCHUNK)
            pltpu.make_async_copy(u_sp.at[src_v.at[i]], rows_v.at[j],
                                  gsems[j]).wait()
            pltpu.sync_copy(rows_v.at[j], acc_sp.at[dst_v.at[i]], add=True)
            pltpu.async_copy(u_sp.at[src_v.at[nxt]], rows_v.at[j], gsems[j])
        return carry

    lax.fori_loop(0, NCHUNK // 4, body, 0)
    for j in range(4):
        pltpu.make_async_copy(u_sp.at[src_v.at[j]], rows_v.at[j],
                              gsems[j]).wait()
    plsc.subcore_barrier()
    pltpu.sync_copy(acc_sp.at[pl.ds(sid * ROWS_W, ROWS_W)],
                    acc_hbm.at[cid, pl.ds(sid * ROWS_W, ROWS_W)])


def _xw_body(xs_ref, xt_ref, w1_ref, xw_ref):
    w1 = w1_ref[...]
    xw_ref[0] = jnp.dot(xs_ref[...], w1, preferred_element_type=jnp.float32)
    xw_ref[1] = jnp.dot(xt_ref[...], w1, preferred_element_type=jnp.float32)


def _tc_xw(x_s_pad, x_t_pad, W1):
    blk = 2048
    grid = NPAD // blk
    return pl.pallas_call(
        _xw_body,
        grid=(grid,),
        in_specs=[
            pl.BlockSpec((blk, D), lambda i: (i, 0)),
            pl.BlockSpec((blk, D), lambda i: (i, 0)),
            pl.BlockSpec((D, H), lambda i: (0, 0)),
        ],
        out_specs=pl.BlockSpec((2, blk, H), lambda i: (0, i, 0)),
        out_shape=jax.ShapeDtypeStruct((2, NPAD, H), jnp.float32),
    )(x_s_pad, x_t_pad, W1)


def _final_body(acc_ref, xw_ref, dinv_ref, b1_ref, batch_ref, w2_ref, b2_ref,
                out_ref):
    embs = []
    for c in range(2):
        a_c = acc_ref[c]
        dv_c = dinv_ref[c][:, None]
        u_c = xw_ref[c] * dv_c
        h = jnp.tanh(dv_c * (a_c + u_c) + b1_ref[...])
        b_c = batch_ref[c]
        iota = lax.broadcasted_iota(jnp.int32, (NPAD, G), 1)
        M = (b_c == iota).astype(jnp.float32)
        sums = lax.dot_general(M, h, (((0,), (0,)), ((), ())),
                               preferred_element_type=jnp.float32)
        ones = jnp.ones((NPAD, 1), jnp.float32)
        cnt = lax.dot_general(M, ones, (((0,), (0,)), ((), ())),
                              preferred_element_type=jnp.float32)
        pooled = sums / jnp.maximum(cnt, 1.0)
        embs.append(jnp.tanh(pooled))
    embedding = jnp.concatenate(embs, axis=1)
    out_ref[...] = (jnp.dot(embedding, w2_ref[...],
                            preferred_element_type=jnp.float32) + b2_ref[...])


def _tc_final(acc, XW2, dinv, b1, batch2, W2, b2):
    return pl.pallas_call(
        _final_body,
        out_shape=jax.ShapeDtypeStruct((G, C), jnp.float32),
    )(acc, XW2, dinv, b1.reshape(1, H), batch2, W2, b2.reshape(1, C))


def _pad_edges(ei, offset):
    src = jnp.concatenate([ei[0], jnp.full((EPAD - E,), N, jnp.int32)])
    dst = jnp.concatenate([ei[1], jnp.full((EPAD - E,), N, jnp.int32)])
    return (src + offset).reshape(16, NCHUNK, CH), dst.reshape(16, NCHUNK, CH)


def kernel(x_s, edge_index_s, x_s_batch, x_t, edge_index_t, x_t_batch, y, W1,
           b1, W2, b2):
    zrows = jnp.zeros((NPAD - N, D), jnp.float32)
    x_s_pad = jnp.concatenate([x_s, zrows])
    x_t_pad = jnp.concatenate([x_t, zrows])

    src_s, dst_s = _pad_edges(edge_index_s, 0)
    src_t, dst_t = _pad_edges(edge_index_t, 0)
    src_comb = jnp.stack([src_s, src_t])
    dst_comb = jnp.stack([dst_s, dst_t])

    bpad = jnp.full((NPAD - N,), G, jnp.int32)
    batch2 = jnp.stack([jnp.concatenate([x_s_batch, bpad]),
                        jnp.concatenate([x_t_batch, bpad])])[..., None]

    deg = _deg_kernel(dst_comb)
    XW2 = _tc_xw(x_s_pad, x_t_pad, W1)
    acc, dinv = _mega_kernel(XW2, src_comb, dst_comb, deg)
    return _tc_final(acc, XW2, dinv, b1, batch2, W2, b2)
